# baseline (speedup 1.0000x reference)
import functools

import jax
import jax.numpy as jnp
from jax import lax
from jax.experimental import pallas as pl
from jax.experimental.pallas import tpu as pltpu
from jax.experimental.pallas import tpu_sc as plsc


def _leaky(v):
    return jnp.where(v >= 0, v, 0.15 * v)


def _he_body1(ea, w1, b1, he1):
    e = ea[...]
    h1 = jnp.dot(e, w1[...], preferred_element_type=jnp.float32,
                 precision=lax.Precision.HIGHEST) + b1[...]
    he1[...] = _leaky(h1)


def _edge_mlp1(ea, w1t, b1, be=3200):
    E, DE = ea.shape
    H = w1t.shape[1]
    grid = E // be
    return pl.pallas_call(
        _he_body1,
        grid=(grid,),
        in_specs=[
            pl.BlockSpec((be, DE), lambda i: (i, 0)),
            pl.BlockSpec((DE, H), lambda i: (0, 0)),
            pl.BlockSpec((1, H), lambda i: (0, 0)),
        ],
        out_specs=pl.BlockSpec((be, H), lambda i: (i, 0)),
        out_shape=jax.ShapeDtypeStruct((E, H), jnp.float32),
    )(ea, w1t, b1)


def _segsum_one(he, dst, npad, with_deg):
    E, H = he.shape
    ns = 16
    half = npad // 2
    arows = half + 32
    zrt = arows // ns
    wrt = half // ns
    grp = 128
    tg = E // grp
    ng = tg // ns
    xg = tg - ng * ns
    zr = zrt // 2

    mesh = plsc.VectorSubcoreMesh(core_axis_name="c", subcore_axis_name="s")
    out_type = [jax.ShapeDtypeStruct((npad, H), jnp.float32)]
    scratch = [
        pltpu.VMEM(((ng + 1) * grp,), jnp.int32),
        pltpu.VMEM((ng + 1, grp), jnp.int32),
        pltpu.VMEM((grp, H), jnp.float32),
        pltpu.VMEM((zr, H), jnp.float32),
        pltpu.VMEM_SHARED((arows, H), jnp.float32),
    ]
    if with_deg:
        out_type.append(jax.ShapeDtypeStruct((npad,), jnp.float32))
        scratch += [
            pltpu.VMEM((grp,), jnp.float32),
            pltpu.VMEM((640,), jnp.float32),
            pltpu.VMEM_SHARED((half + 128,), jnp.float32),
        ]

    @functools.partial(pl.kernel, out_type=out_type, mesh=mesh,
                       scratch_types=scratch)
    def seg(he_h, dst_h, *refs):
        if with_deg:
            (s_o, deg_o, idx_all, idx2_all, rows_a, zbuf, acc,
             ones1, dz, dacc) = refs
        else:
            (s_o, idx_all, idx2_all, rows_a, zbuf, acc) = refs
        cid = lax.axis_index("c")
        sid = lax.axis_index("s")
        zero16 = jnp.zeros((16,), jnp.float32)
        one16 = jnp.ones((16,), jnp.float32)
        lo_c = cid * half
        gb = sid * ng
        xgrp = ns * ng + sid
        has_x = sid < xg

        def zrow(i, _):
            for k in range(H // 16):
                zbuf[i, pl.ds(k * 16, 16)] = zero16
            return 0
        lax.fori_loop(0, zr, zrow, 0)

        if with_deg:
            def frow(i, _):
                dz[pl.ds(i * 16, 16)] = zero16
                return 0
            lax.fori_loop(0, 640 // 16, frow, 0)

            def orow(i, _):
                ones1[pl.ds(i * 16, 16)] = one16
                return 0
            lax.fori_loop(0, grp // 16, orow, 0)

        ich = (ng // 13) * grp

        def iload(i, _):
            pltpu.sync_copy(dst_h.at[pl.ds(gb * grp + i * ich, ich)],
                            idx_all.at[pl.ds(i * ich, ich)])
            return 0
        lax.fori_loop(0, 13, iload, 0)

        @pl.when(has_x)
        def _():
            pltpu.sync_copy(dst_h.at[pl.ds(xgrp * grp, grp)],
                            idx_all.at[pl.ds(ng * grp, grp)])

        def route(i, _):
            for k in range(grp // 16):
                iv = idx_all[pl.ds(i * grp + k * 16, 16)]
                t = iv - lo_c
                m = (t >= 0) & (t < half)
                idx2_all[i, pl.ds(k * 16, 16)] = jnp.where(
                    m, t, half + (iv & 31))
            return 0
        lax.fori_loop(0, ng + 1, route, 0)

        zbase = sid * zrt
        pltpu.sync_copy(zbuf, acc.at[pl.ds(zbase, zr)])
        pltpu.sync_copy(zbuf, acc.at[pl.ds(zbase + zr, zr)])

        if with_deg:
            @pl.when(sid < half // 640)
            def _():
                pltpu.sync_copy(dz, dacc.at[pl.ds(sid * 640, 640)])

            @pl.when(sid == 8)
            def _():
                pltpu.sync_copy(dz.at[pl.ds(0, 128)],
                                dacc.at[pl.ds(half, 128)])

        plsc.subcore_barrier()

        def body(g, _):
            pltpu.sync_copy(he_h.at[pl.ds((gb + g) * grp, grp)], rows_a)
            pltpu.sync_copy(rows_a, acc.at[idx2_all.at[g]], add=True)
            if with_deg:
                pltpu.sync_copy(ones1, dacc.at[idx2_all.at[g]], add=True)
            return 0

        lax.fori_loop(0, ng, body, 0)

        @pl.when(has_x)
        def _():
            pltpu.sync_copy(he_h.at[pl.ds(xgrp * grp, grp)], rows_a)
            pltpu.sync_copy(rows_a, acc.at[idx2_all.at[ng]], add=True)
            if with_deg:
                pltpu.sync_copy(ones1, dacc.at[idx2_all.at[ng]], add=True)

        plsc.subcore_barrier()

        pltpu.sync_copy(acc.at[pl.ds(sid * wrt, wrt)],
                        s_o.at[pl.ds(lo_c + sid * wrt, wrt)])
        if with_deg:
            @pl.when(sid < half // 640)
            def _():
                pltpu.sync_copy(dacc.at[pl.ds(sid * 640, 640)],
                                deg_o.at[pl.ds(lo_c + sid * 640, 640)])

    return seg(he, dst)


def _node_body(x, s1, s2, deg,
               ew2t1, eb2_1, nwxt1, nwet1, nb1,
               ew2t2, eb2_2, nwxt2, nwet2, nb2,
               muwt, mub, lvwt, lvb, mu, lv):
    def dot(a, b):
        return jnp.dot(a, b[...], preferred_element_type=jnp.float32,
                       precision=lax.Precision.HIGHEST)

    d = deg[...]
    u1 = dot(s1[...], ew2t1) + d * eb2_1[...]
    agg1 = d * dot(x[...], nwxt1) + dot(u1, nwet1) + d * nb1[...]
    h1 = _leaky(agg1)
    u2 = dot(s2[...], ew2t2) + d * eb2_2[...]
    agg2 = d * dot(h1, nwxt2) + dot(u2, nwet2) + d * nb2[...]
    h2 = _leaky(agg2)
    mu[...] = dot(h2, muwt) + mub[...]
    lv[...] = dot(h2, lvwt) + lvb[...]


def _node_stage(x, s1, s2, deg, wts, bn=1000):
    N, D = x.shape
    H = s1.shape[1]
    L = wts["muwt"].shape[1]
    grid = N // bn

    def full(shape):
        return pl.BlockSpec(shape, lambda i: (0, 0))

    return pl.pallas_call(
        _node_body,
        grid=(grid,),
        in_specs=[
            pl.BlockSpec((bn, D), lambda i: (i, 0)),
            pl.BlockSpec((bn, H), lambda i: (i, 0)),
            pl.BlockSpec((bn, H), lambda i: (i, 0)),
            pl.BlockSpec((bn, 1), lambda i: (i, 0)),
            full((H, H)), full((1, H)), full((D, H)), full((H, H)),
            full((1, H)),
            full((H, H)), full((1, H)), full((H, H)), full((H, H)),
            full((1, H)),
            full((H, L)), full((1, L)), full((H, L)), full((1, L)),
        ],
        out_specs=[
            pl.BlockSpec((bn, L), lambda i: (i, 0)),
            pl.BlockSpec((bn, L), lambda i: (i, 0)),
        ],
        out_shape=[
            jax.ShapeDtypeStruct((N, L), jnp.float32),
            jax.ShapeDtypeStruct((N, L), jnp.float32),
        ],
    )(x, s1, s2, deg,
      wts["ew2t1"], wts["eb2_1"], wts["nwxt1"], wts["nwet1"], wts["nb1"],
      wts["ew2t2"], wts["eb2_2"], wts["nwxt2"], wts["nwet2"], wts["nb2"],
      wts["muwt"], wts["mub"], wts["lvwt"], wts["lvb"])


def kernel(x, edge_index, edge_attr,
           c1_ew1, c1_eb1, c1_ew2, c1_eb2, c1_nw, c1_nb,
           c2_ew1, c2_eb1, c2_ew2, c2_eb2, c2_nw, c2_nb,
           mu_w, mu_b, lv_w, lv_b):
    N, D = x.shape
    H = c1_ew1.shape[0]
    dst = edge_index[1].astype(jnp.int32)

    npad = ((N + 2047) // 2048) * 2048
    he1 = _edge_mlp1(edge_attr, c1_ew1.T, c1_eb1[None, :])
    s1f, degf = _segsum_one(he1, dst, npad, True)
    he2 = _edge_mlp1(edge_attr, c2_ew1.T, c2_eb1[None, :])
    (s2f,) = _segsum_one(he2, dst, npad, False)
    s1 = s1f[:N]
    s2 = s2f[:N]
    deg = degf[:N][:, None]

    wts = dict(
        ew2t1=c1_ew2.T, eb2_1=c1_eb2[None, :],
        nwxt1=c1_nw[:, :D].T, nwet1=c1_nw[:, D:].T, nb1=c1_nb[None, :],
        ew2t2=c2_ew2.T, eb2_2=c2_eb2[None, :],
        nwxt2=c2_nw[:, :H].T, nwet2=c2_nw[:, H:].T, nb2=c2_nb[None, :],
        muwt=mu_w.T, mub=mu_b[None, :],
        lvwt=lv_w.T, lvb=lv_b[None, :],
    )
    mu, lv = _node_stage(x, s1, s2, deg, wts)
    return mu, lv
